# baseline (speedup 1.0000x reference)
import jax
import jax.numpy as jnp
from jax.experimental import pallas as pl
from jax.experimental.pallas import tpu as pltpu

N = 10000
E = 320000
D = 128
SEED_NUM = 4096
BE = 8000


def _spmm_body(edge_ref, x_ref, acc_ref):
    step = pl.program_id(0)

    @pl.when(step == 0)
    def _init():
        acc_ref[...] = jnp.zeros_like(acc_ref)

    def body(i, carry):
        s = edge_ref[0, 0, i]
        d = edge_ref[0, 1, i]
        acc_ref[pl.ds(d, 1), :] += x_ref[pl.ds(s, 1), :]
        return carry

    jax.lax.fori_loop(0, BE, body, 0)


def _spmm(x_ext, edge_index):
    grid = E // BE
    edges = edge_index.reshape(2, grid, BE).transpose(1, 0, 2)
    return pl.pallas_call(
        _spmm_body,
        grid=(grid,),
        in_specs=[
            pl.BlockSpec((1, 2, BE), lambda i: (i, 0, 0), memory_space=pltpu.SMEM),
            pl.BlockSpec((N, 2 * D), lambda i: (0, 0)),
        ],
        out_specs=pl.BlockSpec((N, 2 * D), lambda i: (0, 0)),
        out_shape=jax.ShapeDtypeStruct((N, 2 * D), jnp.float32),
    )(edges, x_ext)


def _l2n(x, eps=1e-12):
    n = jnp.linalg.norm(x, ord=2, axis=1, keepdims=True)
    return x / jnp.clip(n, eps, None)


def kernel(embeds, edge_index):
    ones_col = jnp.zeros((N, D), jnp.float32).at[:, 0].set(1.0)
    x1 = jnp.concatenate([embeds, ones_col], axis=1)
    h1 = _spmm(x1, edge_index)

    order = h1[:, D:D + 1]
    fst = h1[:, :D] - embeds
    fstN = order

    x2 = jnp.concatenate([fst, jnp.zeros((N, D), jnp.float32).at[:, 0].set(fstN[:, 0])], axis=1)
    h2 = _spmm(x2, edge_index)

    scd = h2[:, :D] - fst - order * embeds
    scdN = h2[:, D:D + 1] - fstN - order

    sub = (fst + scd) / (fstN + scdN + 1e-08)
    sub = _l2n(sub)
    en = _l2n(embeds)
    scores = jax.nn.sigmoid(jnp.sum(sub * en, axis=-1))

    noise = jax.random.uniform(jax.random.key(42), scores.shape, minval=1e-08, maxval=1.0)
    gumbel = -jnp.log(-jnp.log(noise))
    scores = jnp.log(scores) + gumbel

    _, seeds = jax.lax.top_k(scores, SEED_NUM)
    return (scores, seeds)


# speedup vs baseline: 3.6076x; 3.6076x over previous
import functools

import jax
import jax.numpy as jnp
from jax import lax
from jax.experimental import pallas as pl
from jax.experimental.pallas import tpu as pltpu
from jax.experimental.pallas import tpu_sc as plsc

N = 10000
E = 320000
D = 128
SEED_NUM = 4096

NW = 32
ROWS = 313
NP = NW * ROWS
W = 144
ACC = ROWS * W

EB = 20000
NBLK = E // EB
FB = 2048
QCAP = E + 4096
QE = 4096
QB = 128

_MESH = plsc.VectorSubcoreMesh(core_axis_name="c", subcore_axis_name="s")


def _wid():
    return lax.axis_index("c") * 16 + lax.axis_index("s")


@functools.partial(
    pl.kernel,
    out_type=(
        jax.ShapeDtypeStruct((NW, QCAP), jnp.int32),
        jax.ShapeDtypeStruct((NW, QCAP), jnp.int32),
        jax.ShapeDtypeStruct((NW, 16), jnp.int32),
    ),
    mesh=_MESH,
    compiler_params=pltpu.CompilerParams(needs_layout_passes=False),
    scratch_types=[
        pltpu.VMEM((EB,), jnp.int32),
        pltpu.VMEM((EB,), jnp.int32),
        pltpu.VMEM((FB + 32,), jnp.int32),
        pltpu.VMEM((FB + 32,), jnp.int32),
        pltpu.VMEM((16,), jnp.int32),
    ],
)
def _scan(src_hbm, dst_hbm, qsrc_hbm, qdl_hbm, qcnt_hbm, sv, dv, qs_buf, qd_buf, cnt_v):
    wid = _wid()
    base = wid * ROWS

    def mz(i, c):
        qs_buf[pl.ds(i * 16, 16)] = jnp.zeros((16,), jnp.int32)
        qd_buf[pl.ds(i * 16, 16)] = jnp.zeros((16,), jnp.int32)
        return c

    lax.fori_loop(0, (FB + 32) // 16, mz, 0)

    carry = (jnp.int32(0), jnp.int32(0))
    for blk in range(NBLK):
        pltpu.sync_copy(src_hbm.at[pl.ds(blk * EB, EB)], sv)
        pltpu.sync_copy(dst_hbm.at[pl.ds(blk * EB, EB)], dv)

        lane = lax.iota(jnp.int32, 16)

        def body(i, c):
            qpos, nf = c
            d = dv[pl.ds(i * 16, 16)]
            s = sv[pl.ds(i * 16, 16)]
            m = (d >= base) & (d < base + ROWS)
            key = jnp.where(m, lane, 16 + lane)
            _, s_c = plsc.sort_key_val(key, s)
            _, d_c = plsc.sort_key_val(key, d - base)
            qs_buf[pl.ds(qpos, 16)] = s_c
            qd_buf[pl.ds(qpos, 16)] = d_c
            qpos = qpos + plsc.all_reduce_population_count(m)[0]
            full = qpos >= FB

            @pl.when(full)
            def _():
                pltpu.sync_copy(qs_buf.at[pl.ds(0, FB)], qsrc_hbm.at[wid, pl.ds(nf * FB, FB)])
                pltpu.sync_copy(qd_buf.at[pl.ds(0, FB)], qdl_hbm.at[wid, pl.ds(nf * FB, FB)])
                qs_buf[pl.ds(0, 16)] = qs_buf[pl.ds(FB, 16)]
                qd_buf[pl.ds(0, 16)] = qd_buf[pl.ds(FB, 16)]

            qpos = jnp.where(full, qpos - FB, qpos)
            nf = jnp.where(full, nf + 1, nf)
            return (qpos, nf)

        carry = lax.fori_loop(0, EB // 16, body, carry)

    qpos, nf = carry
    pltpu.sync_copy(qs_buf.at[pl.ds(0, FB)], qsrc_hbm.at[wid, pl.ds(nf * FB, FB)])
    pltpu.sync_copy(qd_buf.at[pl.ds(0, FB)], qdl_hbm.at[wid, pl.ds(nf * FB, FB)])
    cnt_v[...] = jnp.full((16,), nf * FB + qpos, jnp.int32)
    pltpu.sync_copy(cnt_v, qcnt_hbm.at[wid])


@functools.partial(
    pl.kernel,
    out_type=jax.ShapeDtypeStruct((NP * W,), jnp.float32),
    mesh=_MESH,
    compiler_params=pltpu.CompilerParams(needs_layout_passes=False, use_tc_tiling_on_sc=False),
    scratch_types=[
        pltpu.VMEM((ACC,), jnp.float32),
        pltpu.VMEM((QE + 16,), jnp.int32),
        pltpu.VMEM((QE + 16,), jnp.int32),
        pltpu.VMEM((QB, W), jnp.float32),
        pltpu.VMEM((QB, W), jnp.float32),
        pltpu.VMEM((NW * 16,), jnp.int32),
        pltpu.SemaphoreType.DMA,
        pltpu.SemaphoreType.DMA,
    ],
)
def _hop(x_hbm, qsrc_hbm, qdl_hbm, qcnt_hbm, out_hbm,
         acc, qs_v, qd_v, rows0, rows1, cnt_all, sem0, sem1):
    wid = _wid()
    pltpu.sync_copy(qcnt_hbm, cnt_all)
    cnt = cnt_all[pl.ds(wid * 16, 16)][0]

    def z(i, c):
        acc[pl.ds(i * 16, 16)] = jnp.zeros((16,), jnp.float32)
        return c

    lax.fori_loop(0, ACC // 16, z, 0)

    def start(k, rows, sem):
        pltpu.async_copy(x_hbm.at[qs_v.at[pl.ds(k * QB, QB)]], rows, sem)

    def wait(rows, sem):
        pltpu.make_async_copy(x_hbm.at[pl.ds(0, QB)], rows, sem).wait()

    def accumulate(rows, eoff, n_e):
        def ebody(e, c):
            b = qd_v[pl.ds(eoff + e, 16)][0] * W
            for j in range(W // 16):
                acc[pl.ds(b + j * 16, 16)] += rows[e, pl.ds(j * 16, 16)]
            return c

        lax.fori_loop(0, n_e, ebody, 0)

    nchunks = lax.div(cnt + (QE - 1), QE)

    def chunk_body(c, carry):
        coff = c * QE
        pltpu.sync_copy(qsrc_hbm.at[wid, pl.ds(coff, QE)], qs_v.at[pl.ds(0, QE)])
        pltpu.sync_copy(qdl_hbm.at[wid, pl.ds(coff, QE)], qd_v.at[pl.ds(0, QE)])

        def _san(i, sc):
            v = qs_v[pl.ds(i * 16, 16)]
            qs_v[pl.ds(i * 16, 16)] = jnp.minimum(jnp.maximum(v, 0), NP - 1)
            return sc

        lax.fori_loop(0, QE // 16, _san, 0)
        m_edges = jnp.minimum(QE, cnt - coff)
        nb = lax.div(m_edges + (QB - 1), QB)

        @pl.when(nb > 0)
        def _():
            start(0, rows0, sem0)

        @pl.when(nb > 1)
        def _():
            start(1, rows1, sem1)

        def blk_body(k, kc):
            eoff = k * QB
            n_e = jnp.minimum(QB, m_edges - eoff)

            @pl.when(k % 2 == 0)
            def _():
                wait(rows0, sem0)
                accumulate(rows0, eoff, n_e)

                @pl.when(k + 2 < nb)
                def _():
                    start(k + 2, rows0, sem0)

            @pl.when(k % 2 == 1)
            def _():
                wait(rows1, sem1)
                accumulate(rows1, eoff, n_e)

                @pl.when(k + 2 < nb)
                def _():
                    start(k + 2, rows1, sem1)

            return kc

        lax.fori_loop(0, nb, blk_body, 0)
        return carry

    lax.fori_loop(0, nchunks, chunk_body, 0)
    pltpu.sync_copy(acc, out_hbm.at[pl.ds(wid * ACC, ACC)])


def _l2n(x, eps=1e-12):
    n = jnp.linalg.norm(x, ord=2, axis=1, keepdims=True)
    return x / jnp.clip(n, eps, None)


def kernel(embeds, edge_index):

    qsrc, qdl, qcnt = _scan(edge_index[0], edge_index[1])
    qcnt = qcnt.reshape(NW * 16)

    pad = jnp.zeros((NP - N, D), jnp.float32)
    onecol = jnp.zeros((NP, W - D), jnp.float32).at[:N, 0].set(1.0)
    x1 = jnp.concatenate([jnp.concatenate([embeds, pad], 0), onecol], 1)

    h1 = _hop(x1, qsrc, qdl, qcnt).reshape(NP, W)

    order = h1[:N, D:D + 1]
    fst = h1[:N, :D] - embeds
    fstN = order

    degcol = jnp.zeros((NP, W - D), jnp.float32).at[:N, 0].set(fstN[:, 0])
    x2 = jnp.concatenate([jnp.concatenate([fst, pad], 0), degcol], 1)

    h2 = _hop(x2, qsrc, qdl, qcnt).reshape(NP, W)

    scd = h2[:N, :D] - fst - order * embeds
    scdN = h2[:N, D:D + 1] - fstN - order

    sub = (fst + scd) / (fstN + scdN + 1e-08)
    sub = _l2n(sub)
    en = _l2n(embeds)
    scores = jax.nn.sigmoid(jnp.sum(sub * en, axis=-1))

    noise = jax.random.uniform(jax.random.key(42), scores.shape, minval=1e-08, maxval=1.0)
    gumbel = -jnp.log(-jnp.log(noise))
    scores = jnp.log(scores) + gumbel

    _, seeds = jax.lax.top_k(scores, SEED_NUM)
    return (scores, seeds)


# speedup vs baseline: 7.6319x; 2.1155x over previous
import functools

import jax
import jax.numpy as jnp
from jax import lax
from jax.experimental import pallas as pl
from jax.experimental.pallas import tpu as pltpu
from jax.experimental.pallas import tpu_sc as plsc

N = 10000
E = 320000
D = 128
SEED_NUM = 4096

NW = 32
ROWS = 313
NP = NW * ROWS
W = 144
SHIFT = 9

EB = 20000
NBLK = E // EB
FB = 2048
QE = 4096
QB = 128
NBQ = QE // QB
QCAP = E + QE
SROWS = 16 * ROWS

_MESH = plsc.VectorSubcoreMesh(core_axis_name="c", subcore_axis_name="s")


def _wid():
    return lax.axis_index("c") * 16 + lax.axis_index("s")


@functools.partial(
    pl.kernel,
    out_type=(
        jax.ShapeDtypeStruct((NW, QCAP), jnp.int32),
        jax.ShapeDtypeStruct((NW, 16), jnp.int32),
    ),
    mesh=_MESH,
    compiler_params=pltpu.CompilerParams(needs_layout_passes=False),
    scratch_types=[
        pltpu.VMEM((EB,), jnp.int32),
        pltpu.VMEM((EB,), jnp.int32),
        pltpu.VMEM((FB + 32,), jnp.int32),
        pltpu.VMEM((16,), jnp.int32),
    ],
)
def _scan(src_hbm, dst_hbm, qpk_hbm, qcnt_hbm, sv, dv, qp_buf, cnt_v):
    wid = _wid()
    base = wid * ROWS

    def mz(i, c):
        qp_buf[pl.ds(i * 16, 16)] = jnp.zeros((16,), jnp.int32)
        return c

    lax.fori_loop(0, (FB + 32) // 16, mz, 0)

    carry = (jnp.int32(0), jnp.int32(0))
    for blk in range(NBLK):
        pltpu.sync_copy(src_hbm.at[pl.ds(blk * EB, EB)], sv)
        pltpu.sync_copy(dst_hbm.at[pl.ds(blk * EB, EB)], dv)

        lane = lax.iota(jnp.int32, 16)

        def body(i, c):
            qpos, nf = c
            d = dv[pl.ds(i * 16, 16)]
            s = sv[pl.ds(i * 16, 16)]
            m = (d >= base) & (d < base + ROWS)
            key = jnp.where(m, lane, 16 + lane)
            packed = jnp.bitwise_or(lax.shift_left(s, SHIFT), d - base)
            _, p_c = plsc.sort_key_val(key, packed)
            qp_buf[pl.ds(qpos, 16)] = p_c
            qpos = qpos + plsc.all_reduce_population_count(m)[0]
            full = qpos >= FB

            @pl.when(full)
            def _():
                pltpu.sync_copy(qp_buf.at[pl.ds(0, FB)], qpk_hbm.at[wid, pl.ds(nf * FB, FB)])
                qp_buf[pl.ds(0, 16)] = qp_buf[pl.ds(FB, 16)]

            qpos = jnp.where(full, qpos - FB, qpos)
            nf = jnp.where(full, nf + 1, nf)
            return (qpos, nf)

        carry = lax.fori_loop(0, EB // 16, body, carry)

    qpos, nf = carry
    pltpu.sync_copy(qp_buf.at[pl.ds(0, FB)], qpk_hbm.at[wid, pl.ds(nf * FB, FB)])
    cnt_v[...] = jnp.full((16,), nf * FB + qpos, jnp.int32)
    pltpu.sync_copy(cnt_v, qcnt_hbm.at[wid])


@functools.partial(
    pl.kernel,
    out_type=jax.ShapeDtypeStruct((NP, W), jnp.float32),
    mesh=_MESH,
    compiler_params=pltpu.CompilerParams(needs_layout_passes=False, use_tc_tiling_on_sc=False),
    scratch_types=[
        pltpu.VMEM((QE + 16,), jnp.int32),
        pltpu.VMEM((QE,), jnp.int32),
        pltpu.VMEM((NBQ, QB), jnp.int32),
        pltpu.VMEM((QB, W), jnp.float32),
        pltpu.VMEM((QB, W), jnp.float32),
        pltpu.VMEM((NW * 16,), jnp.int32),
        pltpu.VMEM_SHARED((SROWS + 8, W), jnp.float32),
        pltpu.SemaphoreType.DMA,
        pltpu.SemaphoreType.DMA,
    ],
)
def _hop(x_hbm, qpk_hbm, qcnt_hbm, out_hbm,
         qp_v, qs_v, qd_v, rows0, rows1, cnt_all, sacc, sem0, sem1):
    cid = lax.axis_index("c")
    sid = lax.axis_index("s")
    wid = cid * 16 + sid
    trash = SROWS + jnp.mod(sid, 8)

    pltpu.sync_copy(qcnt_hbm, cnt_all)
    cnt = cnt_all[pl.ds(wid * 16, 16)][0]

    def z(i, c):
        for j in range(W // 16):
            rows0[i, pl.ds(j * 16, 16)] = jnp.zeros((16,), jnp.float32)
        return c

    lax.fori_loop(0, QB, z, 0)
    pltpu.sync_copy(rows0, sacc.at[pl.ds(sid * ROWS, QB)])
    pltpu.sync_copy(rows0, sacc.at[pl.ds(sid * ROWS + QB, QB)])
    pltpu.sync_copy(rows0.at[pl.ds(0, ROWS - 2 * QB)], sacc.at[pl.ds(sid * ROWS + 2 * QB, ROWS - 2 * QB)])

    lane = lax.iota(jnp.int32, 16)

    def start(k, rows, sem):
        pltpu.async_copy(x_hbm.at[qs_v.at[pl.ds(k * QB, QB)]], rows, sem)

    def wait(rows, sem):
        pltpu.make_async_copy(x_hbm.at[pl.ds(0, QB)], rows, sem).wait()

    nchunks = lax.div(cnt + (QE - 1), QE)

    def chunk_body(c, carry):
        coff = c * QE
        pltpu.sync_copy(qpk_hbm.at[wid, pl.ds(coff, QE)], qp_v.at[pl.ds(0, QE)])

        def unpack(i, sc):
            p = qp_v[pl.ds(i * 16, 16)]
            src = lax.shift_right_logical(p, SHIFT)
            src = jnp.minimum(src, NP - 1)
            dl = jnp.bitwise_and(p, (1 << SHIFT) - 1)
            valid = (coff + i * 16 + lane) < cnt
            qs_v[pl.ds(i * 16, 16)] = src
            sidx = jnp.where(valid, sid * ROWS + dl, trash)
            qd_v[lax.div(i, 8), pl.ds(jnp.mod(i, 8) * 16, 16)] = sidx
            return sc

        lax.fori_loop(0, QE // 16, unpack, 0)

        m_edges = jnp.minimum(QE, cnt - coff)
        nb = lax.div(m_edges + (QB - 1), QB)

        @pl.when(nb > 0)
        def _():
            start(0, rows0, sem0)

        @pl.when(nb > 1)
        def _():
            start(1, rows1, sem1)

        def blk_body(k, kc):
            @pl.when(k % 2 == 0)
            def _():
                wait(rows0, sem0)
                pltpu.sync_copy(rows0, sacc.at[qd_v.at[k]], add=True)

                @pl.when(k + 2 < nb)
                def _():
                    start(k + 2, rows0, sem0)

            @pl.when(k % 2 == 1)
            def _():
                wait(rows1, sem1)
                pltpu.sync_copy(rows1, sacc.at[qd_v.at[k]], add=True)

                @pl.when(k + 2 < nb)
                def _():
                    start(k + 2, rows1, sem1)

            return kc

        lax.fori_loop(0, nb, blk_body, 0)
        return carry

    lax.fori_loop(0, nchunks, chunk_body, 0)
    pltpu.sync_copy(sacc.at[pl.ds(sid * ROWS, ROWS)], out_hbm.at[pl.ds(wid * ROWS, ROWS)])


def _l2n(x, eps=1e-12):
    n = jnp.linalg.norm(x, ord=2, axis=1, keepdims=True)
    return x / jnp.clip(n, eps, None)


def kernel(embeds, edge_index):
    qpk, qcnt = _scan(edge_index[0], edge_index[1])
    qcnt = qcnt.reshape(NW * 16)

    pad = jnp.zeros((NP - N, D), jnp.float32)
    onecol = jnp.zeros((NP, W - D), jnp.float32).at[:N, 0].set(1.0)
    x1 = jnp.concatenate([jnp.concatenate([embeds, pad], 0), onecol], 1)

    h1 = _hop(x1, qpk, qcnt)

    order = h1[:N, D:D + 1]
    fst = h1[:N, :D] - embeds
    fstN = order

    degcol = jnp.zeros((NP, W - D), jnp.float32).at[:N, 0].set(fstN[:, 0])
    x2 = jnp.concatenate([jnp.concatenate([fst, pad], 0), degcol], 1)

    h2 = _hop(x2, qpk, qcnt)

    scd = h2[:N, :D] - fst - order * embeds
    scdN = h2[:N, D:D + 1] - fstN - order

    sub = (fst + scd) / (fstN + scdN + 1e-08)
    sub = _l2n(sub)
    en = _l2n(embeds)
    scores = jax.nn.sigmoid(jnp.sum(sub * en, axis=-1))

    noise = jax.random.uniform(jax.random.key(42), scores.shape, minval=1e-08, maxval=1.0)
    gumbel = -jnp.log(-jnp.log(noise))
    scores = jnp.log(scores) + gumbel

    _, seeds = jax.lax.top_k(scores, SEED_NUM)
    return (scores, seeds)


# speedup vs baseline: 9.3726x; 1.2281x over previous
import functools

import jax
import jax.numpy as jnp
from jax import lax
from jax.experimental import pallas as pl
from jax.experimental.pallas import tpu as pltpu
from jax.experimental.pallas import tpu_sc as plsc

N = 10000
E = 320000
D = 128
SEED_NUM = 4096

NW = 32
ROWS = 313
NP = NW * ROWS
W = 144
SHIFT = 9

EB = 16000
NBLK = E // EB
UNR = 8
FB = 2048
QE = 4096
QB = 128
NBQ = QE // QB
QCAP = E + QE
SROWS = 16 * ROWS

_MESH = plsc.VectorSubcoreMesh(core_axis_name="c", subcore_axis_name="s")


def _wid():
    return lax.axis_index("c") * 16 + lax.axis_index("s")


@functools.partial(
    pl.kernel,
    out_type=(
        jax.ShapeDtypeStruct((NW, QCAP), jnp.int32),
        jax.ShapeDtypeStruct((NW, 16), jnp.int32),
    ),
    mesh=_MESH,
    compiler_params=pltpu.CompilerParams(needs_layout_passes=False),
    scratch_types=[
        pltpu.VMEM((EB,), jnp.int32),
        pltpu.VMEM((EB,), jnp.int32),
        pltpu.VMEM((FB + 16 * UNR,), jnp.int32),
        pltpu.VMEM((16,), jnp.int32),
    ],
)
def _scan(src_hbm, dst_hbm, qpk_hbm, qcnt_hbm, sv, dv, qp_buf, cnt_v):
    wid = _wid()
    base = wid * ROWS

    def mz(i, c):
        qp_buf[pl.ds(i * 16, 16)] = jnp.zeros((16,), jnp.int32)
        return c

    lax.fori_loop(0, (FB + 16 * UNR) // 16, mz, 0)

    carry = (jnp.int32(0), jnp.int32(0))
    for blk in range(NBLK):
        pltpu.sync_copy(src_hbm.at[pl.ds(blk * EB, EB)], sv)
        pltpu.sync_copy(dst_hbm.at[pl.ds(blk * EB, EB)], dv)

        lane = lax.iota(jnp.int32, 16)

        def body(i, c):
            qpos, nf = c
            for u in range(UNR):
                off = i * (16 * UNR) + u * 16
                d = dv[pl.ds(off, 16)]
                s = sv[pl.ds(off, 16)]
                m = (d >= base) & (d < base + ROWS)
                key = jnp.where(m, lane, 16 + lane)
                packed = jnp.bitwise_or(lax.shift_left(s, SHIFT), d - base)
                _, p_c = plsc.sort_key_val(key, packed)
                qp_buf[pl.ds(qpos, 16)] = p_c
                qpos = qpos + plsc.all_reduce_population_count(m)[0]
            full = qpos >= FB

            @pl.when(full)
            def _():
                pltpu.sync_copy(qp_buf.at[pl.ds(0, FB)], qpk_hbm.at[wid, pl.ds(nf * FB, FB)])
                for u in range(UNR):
                    qp_buf[pl.ds(u * 16, 16)] = qp_buf[pl.ds(FB + u * 16, 16)]

            qpos = jnp.where(full, qpos - FB, qpos)
            nf = jnp.where(full, nf + 1, nf)
            return (qpos, nf)

        carry = lax.fori_loop(0, EB // (16 * UNR), body, carry)

    qpos, nf = carry
    pltpu.sync_copy(qp_buf.at[pl.ds(0, FB)], qpk_hbm.at[wid, pl.ds(nf * FB, FB)])
    cnt_v[...] = jnp.full((16,), nf * FB + qpos, jnp.int32)
    pltpu.sync_copy(cnt_v, qcnt_hbm.at[wid])


@functools.partial(
    pl.kernel,
    out_type=jax.ShapeDtypeStruct((NP, W), jnp.float32),
    mesh=_MESH,
    compiler_params=pltpu.CompilerParams(needs_layout_passes=False, use_tc_tiling_on_sc=False),
    scratch_types=[
        pltpu.VMEM((QE + 16,), jnp.int32),
        pltpu.VMEM((QE,), jnp.int32),
        pltpu.VMEM((NBQ, QB), jnp.int32),
        pltpu.VMEM((QB, W), jnp.float32),
        pltpu.VMEM((QB, W), jnp.float32),
        pltpu.VMEM((NW * 16,), jnp.int32),
        pltpu.VMEM_SHARED((SROWS + 8, W), jnp.float32),
        pltpu.SemaphoreType.DMA,
        pltpu.SemaphoreType.DMA,
    ],
)
def _hop(x_hbm, qpk_hbm, qcnt_hbm, out_hbm,
         qp_v, qs_v, qd_v, rows0, rows1, cnt_all, sacc, sem0, sem1):
    cid = lax.axis_index("c")
    sid = lax.axis_index("s")
    wid = cid * 16 + sid
    trash = SROWS + jnp.mod(sid, 8)

    pltpu.sync_copy(qcnt_hbm, cnt_all)
    cnt = cnt_all[pl.ds(wid * 16, 16)][0]

    def z(i, c):
        for j in range(W // 16):
            rows0[i, pl.ds(j * 16, 16)] = jnp.zeros((16,), jnp.float32)
        return c

    lax.fori_loop(0, QB, z, 0)
    pltpu.sync_copy(rows0, sacc.at[pl.ds(sid * ROWS, QB)])
    pltpu.sync_copy(rows0, sacc.at[pl.ds(sid * ROWS + QB, QB)])
    pltpu.sync_copy(rows0.at[pl.ds(0, ROWS - 2 * QB)], sacc.at[pl.ds(sid * ROWS + 2 * QB, ROWS - 2 * QB)])

    lane = lax.iota(jnp.int32, 16)

    def start(k, rows, sem):
        pltpu.async_copy(x_hbm.at[qs_v.at[pl.ds(k * QB, QB)]], rows, sem)

    def wait(rows, sem):
        pltpu.make_async_copy(x_hbm.at[pl.ds(0, QB)], rows, sem).wait()

    nchunks = lax.div(cnt + (QE - 1), QE)

    def chunk_body(c, carry):
        coff = c * QE
        pltpu.sync_copy(qpk_hbm.at[wid, pl.ds(coff, QE)], qp_v.at[pl.ds(0, QE)])

        def unpack(i, sc):
            p = qp_v[pl.ds(i * 16, 16)]
            src = lax.shift_right_logical(p, SHIFT)
            src = jnp.minimum(src, NP - 1)
            dl = jnp.bitwise_and(p, (1 << SHIFT) - 1)
            valid = (coff + i * 16 + lane) < cnt
            qs_v[pl.ds(i * 16, 16)] = src
            sidx = jnp.where(valid, sid * ROWS + dl, trash)
            qd_v[lax.div(i, 8), pl.ds(jnp.mod(i, 8) * 16, 16)] = sidx
            return sc

        lax.fori_loop(0, QE // 16, unpack, 0)

        m_edges = jnp.minimum(QE, cnt - coff)
        nb = lax.div(m_edges + (QB - 1), QB)

        @pl.when(nb > 0)
        def _():
            start(0, rows0, sem0)

        @pl.when(nb > 1)
        def _():
            start(1, rows1, sem1)

        def blk_body(k, kc):
            @pl.when(k % 2 == 0)
            def _():
                wait(rows0, sem0)
                pltpu.sync_copy(rows0, sacc.at[qd_v.at[k]], add=True)

                @pl.when(k + 2 < nb)
                def _():
                    start(k + 2, rows0, sem0)

            @pl.when(k % 2 == 1)
            def _():
                wait(rows1, sem1)
                pltpu.sync_copy(rows1, sacc.at[qd_v.at[k]], add=True)

                @pl.when(k + 2 < nb)
                def _():
                    start(k + 2, rows1, sem1)

            return kc

        lax.fori_loop(0, nb, blk_body, 0)
        return carry

    lax.fori_loop(0, nchunks, chunk_body, 0)
    pltpu.sync_copy(sacc.at[pl.ds(sid * ROWS, ROWS)], out_hbm.at[pl.ds(wid * ROWS, ROWS)])


def _l2n(x, eps=1e-12):
    n = jnp.linalg.norm(x, ord=2, axis=1, keepdims=True)
    return x / jnp.clip(n, eps, None)


def kernel(embeds, edge_index):
    qpk, qcnt = _scan(edge_index[0], edge_index[1])
    qcnt = qcnt.reshape(NW * 16)

    pad = jnp.zeros((NP - N, D), jnp.float32)
    onecol = jnp.zeros((NP, W - D), jnp.float32).at[:N, 0].set(1.0)
    x1 = jnp.concatenate([jnp.concatenate([embeds, pad], 0), onecol], 1)

    h1 = _hop(x1, qpk, qcnt)

    order = h1[:N, D:D + 1]
    fst = h1[:N, :D] - embeds
    fstN = order

    degcol = jnp.zeros((NP, W - D), jnp.float32).at[:N, 0].set(fstN[:, 0])
    x2 = jnp.concatenate([jnp.concatenate([fst, pad], 0), degcol], 1)

    h2 = _hop(x2, qpk, qcnt)

    scd = h2[:N, :D] - fst - order * embeds
    scdN = h2[:N, D:D + 1] - fstN - order

    sub = (fst + scd) / (fstN + scdN + 1e-08)
    sub = _l2n(sub)
    en = _l2n(embeds)
    scores = jax.nn.sigmoid(jnp.sum(sub * en, axis=-1))

    noise = jax.random.uniform(jax.random.key(42), scores.shape, minval=1e-08, maxval=1.0)
    gumbel = -jnp.log(-jnp.log(noise))
    scores = jnp.log(scores) + gumbel

    _, seeds = jax.lax.top_k(scores, SEED_NUM)
    return (scores, seeds)


# speedup vs baseline: 9.4368x; 1.0069x over previous
import functools

import jax
import jax.numpy as jnp
from jax import lax
from jax.experimental import pallas as pl
from jax.experimental.pallas import tpu as pltpu
from jax.experimental.pallas import tpu_sc as plsc

N = 10000
E = 320000
D = 128
SEED_NUM = 4096

NW = 32
ROWS = 313
NP = NW * ROWS
W = 144
SHIFT = 9

EB = 12800
NBLK = E // EB
UNR = 16
FB = 2048
QE = 4096
QB = 128
NBQ = QE // QB
QCAP = E + QE
SROWS = 16 * ROWS

_MESH = plsc.VectorSubcoreMesh(core_axis_name="c", subcore_axis_name="s")


def _wid():
    return lax.axis_index("c") * 16 + lax.axis_index("s")


@functools.partial(
    pl.kernel,
    out_type=(
        jax.ShapeDtypeStruct((NW, QCAP), jnp.int32),
        jax.ShapeDtypeStruct((NW, 16), jnp.int32),
    ),
    mesh=_MESH,
    compiler_params=pltpu.CompilerParams(needs_layout_passes=False),
    scratch_types=[
        pltpu.VMEM((EB,), jnp.int32),
        pltpu.VMEM((EB,), jnp.int32),
        pltpu.VMEM((FB + 16 * UNR,), jnp.int32),
        pltpu.VMEM((16,), jnp.int32),
    ],
)
def _scan(src_hbm, dst_hbm, qpk_hbm, qcnt_hbm, sv, dv, qp_buf, cnt_v):
    wid = _wid()
    base = wid * ROWS

    def mz(i, c):
        qp_buf[pl.ds(i * 16, 16)] = jnp.zeros((16,), jnp.int32)
        return c

    lax.fori_loop(0, (FB + 16 * UNR) // 16, mz, 0)

    carry = (jnp.int32(0), jnp.int32(0))
    for blk in range(NBLK):
        pltpu.sync_copy(src_hbm.at[pl.ds(blk * EB, EB)], sv)
        pltpu.sync_copy(dst_hbm.at[pl.ds(blk * EB, EB)], dv)

        lane = lax.iota(jnp.int32, 16)

        def body(i, c):
            qpos, nf = c
            for u in range(UNR):
                off = i * (16 * UNR) + u * 16
                d = dv[pl.ds(off, 16)]
                s = sv[pl.ds(off, 16)]
                m = (d >= base) & (d < base + ROWS)
                key = jnp.where(m, lane, 16 + lane)
                packed = jnp.bitwise_or(lax.shift_left(s, SHIFT), d - base)
                _, p_c = plsc.sort_key_val(key, packed)
                qp_buf[pl.ds(qpos, 16)] = p_c
                qpos = qpos + plsc.all_reduce_population_count(m)[0]
            full = qpos >= FB

            @pl.when(full)
            def _():
                pltpu.sync_copy(qp_buf.at[pl.ds(0, FB)], qpk_hbm.at[wid, pl.ds(nf * FB, FB)])
                for u in range(UNR):
                    qp_buf[pl.ds(u * 16, 16)] = qp_buf[pl.ds(FB + u * 16, 16)]

            qpos = jnp.where(full, qpos - FB, qpos)
            nf = jnp.where(full, nf + 1, nf)
            return (qpos, nf)

        carry = lax.fori_loop(0, EB // (16 * UNR), body, carry)

    qpos, nf = carry
    pltpu.sync_copy(qp_buf.at[pl.ds(0, FB)], qpk_hbm.at[wid, pl.ds(nf * FB, FB)])
    cnt_v[...] = jnp.full((16,), nf * FB + qpos, jnp.int32)
    pltpu.sync_copy(cnt_v, qcnt_hbm.at[wid])


@functools.partial(
    pl.kernel,
    out_type=jax.ShapeDtypeStruct((NP, W), jnp.float32),
    mesh=_MESH,
    compiler_params=pltpu.CompilerParams(needs_layout_passes=False, use_tc_tiling_on_sc=False),
    scratch_types=[
        pltpu.VMEM((QE + 16,), jnp.int32),
        pltpu.VMEM((QE,), jnp.int32),
        pltpu.VMEM((NBQ, QB), jnp.int32),
        pltpu.VMEM((QB, W), jnp.float32),
        pltpu.VMEM((QB, W), jnp.float32),
        pltpu.VMEM((NW * 16,), jnp.int32),
        pltpu.VMEM_SHARED((SROWS + 8, W), jnp.float32),
        pltpu.SemaphoreType.DMA,
        pltpu.SemaphoreType.DMA,
    ],
)
def _hop(x_hbm, qpk_hbm, qcnt_hbm, out_hbm,
         qp_v, qs_v, qd_v, rows0, rows1, cnt_all, sacc, sem0, sem1):
    cid = lax.axis_index("c")
    sid = lax.axis_index("s")
    wid = cid * 16 + sid
    trash = SROWS + jnp.mod(sid, 8)

    pltpu.sync_copy(qcnt_hbm, cnt_all)
    cnt = cnt_all[pl.ds(wid * 16, 16)][0]

    def z(i, c):
        for j in range(W // 16):
            rows0[i, pl.ds(j * 16, 16)] = jnp.zeros((16,), jnp.float32)
        return c

    lax.fori_loop(0, QB, z, 0)
    pltpu.sync_copy(rows0, sacc.at[pl.ds(sid * ROWS, QB)])
    pltpu.sync_copy(rows0, sacc.at[pl.ds(sid * ROWS + QB, QB)])
    pltpu.sync_copy(rows0.at[pl.ds(0, ROWS - 2 * QB)], sacc.at[pl.ds(sid * ROWS + 2 * QB, ROWS - 2 * QB)])

    lane = lax.iota(jnp.int32, 16)

    def start(k, rows, sem):
        pltpu.async_copy(x_hbm.at[qs_v.at[pl.ds(k * QB, QB)]], rows, sem)

    def wait(rows, sem):
        pltpu.make_async_copy(x_hbm.at[pl.ds(0, QB)], rows, sem).wait()

    nchunks = lax.div(cnt + (QE - 1), QE)

    def chunk_body(c, carry):
        coff = c * QE
        pltpu.sync_copy(qpk_hbm.at[wid, pl.ds(coff, QE)], qp_v.at[pl.ds(0, QE)])

        def unpack(i, sc):
            p = qp_v[pl.ds(i * 16, 16)]
            src = lax.shift_right_logical(p, SHIFT)
            src = jnp.minimum(src, NP - 1)
            dl = jnp.bitwise_and(p, (1 << SHIFT) - 1)
            valid = (coff + i * 16 + lane) < cnt
            qs_v[pl.ds(i * 16, 16)] = src
            sidx = jnp.where(valid, sid * ROWS + dl, trash)
            qd_v[lax.div(i, 8), pl.ds(jnp.mod(i, 8) * 16, 16)] = sidx
            return sc

        lax.fori_loop(0, QE // 16, unpack, 0)

        m_edges = jnp.minimum(QE, cnt - coff)
        nb = lax.div(m_edges + (QB - 1), QB)

        @pl.when(nb > 0)
        def _():
            start(0, rows0, sem0)

        @pl.when(nb > 1)
        def _():
            start(1, rows1, sem1)

        def blk_body(k, kc):
            @pl.when(k % 2 == 0)
            def _():
                wait(rows0, sem0)
                pltpu.sync_copy(rows0, sacc.at[qd_v.at[k]], add=True)

                @pl.when(k + 2 < nb)
                def _():
                    start(k + 2, rows0, sem0)

            @pl.when(k % 2 == 1)
            def _():
                wait(rows1, sem1)
                pltpu.sync_copy(rows1, sacc.at[qd_v.at[k]], add=True)

                @pl.when(k + 2 < nb)
                def _():
                    start(k + 2, rows1, sem1)

            return kc

        lax.fori_loop(0, nb, blk_body, 0)
        return carry

    lax.fori_loop(0, nchunks, chunk_body, 0)
    pltpu.sync_copy(sacc.at[pl.ds(sid * ROWS, ROWS)], out_hbm.at[pl.ds(wid * ROWS, ROWS)])


def _l2n(x, eps=1e-12):
    n = jnp.linalg.norm(x, ord=2, axis=1, keepdims=True)
    return x / jnp.clip(n, eps, None)


def kernel(embeds, edge_index):
    qpk, qcnt = _scan(edge_index[0], edge_index[1])
    qcnt = qcnt.reshape(NW * 16)

    pad = jnp.zeros((NP - N, D), jnp.float32)
    onecol = jnp.zeros((NP, W - D), jnp.float32).at[:N, 0].set(1.0)
    x1 = jnp.concatenate([jnp.concatenate([embeds, pad], 0), onecol], 1)

    h1 = _hop(x1, qpk, qcnt)

    order = h1[:N, D:D + 1]
    fst = h1[:N, :D] - embeds
    fstN = order

    degcol = jnp.zeros((NP, W - D), jnp.float32).at[:N, 0].set(fstN[:, 0])
    x2 = jnp.concatenate([jnp.concatenate([fst, pad], 0), degcol], 1)

    h2 = _hop(x2, qpk, qcnt)

    scd = h2[:N, :D] - fst - order * embeds
    scdN = h2[:N, D:D + 1] - fstN - order

    sub = (fst + scd) / (fstN + scdN + 1e-08)
    sub = _l2n(sub)
    en = _l2n(embeds)
    scores = jax.nn.sigmoid(jnp.sum(sub * en, axis=-1))

    noise = jax.random.uniform(jax.random.key(42), scores.shape, minval=1e-08, maxval=1.0)
    gumbel = -jnp.log(-jnp.log(noise))
    scores = jnp.log(scores) + gumbel

    _, seeds = jax.lax.top_k(scores, SEED_NUM)
    return (scores, seeds)
